# baseline (speedup 1.0000x reference)
import functools

import jax
import jax.numpy as jnp
from jax import lax
from jax.experimental import pallas as pl
from jax.experimental.pallas import tpu as pltpu
from jax.experimental.pallas import tpu_sc as plsc

N = 10000
E = 320000
F = 128
G = 128
L = 3
EPS_BN = 1e-5

NC = 2
NS = 16
NW = NC * NS
CH = 64
EPW = 10240
CHUNKS = EPW // CH
CPG = 16
GROUPS = CHUNKS // CPG
E_PAD = NW * EPW
N_PAD = 10112
RPT = N_PAD // NS
GD = 3
ED = 2

BE = 2048
BN_ = 400
NB = N // BN_


E8 = E // 8
BE8 = 320

import numpy as _np
_PERM = _np.empty(128, _np.int32)
for _p in range(128):
    _k, _i = _p // 32, _p % 32
    _PERM[_p] = 32 * _k + (_i // 2 if _i % 2 == 0 else 16 + _i // 2)


def _edge_mm_body(a_ref, w_ref, b_ref, o_ref):
    a = a_ref[...]
    o_ref[...] = (jnp.dot(a, w_ref[...], preferred_element_type=jnp.float32)
                  + b_ref[...]).astype(jnp.bfloat16)


def _edge_matmul(edge_attr8, Wbig, bbig):
    return pl.pallas_call(
        _edge_mm_body,
        grid=(E8 // BE8,),
        in_specs=[
            pl.BlockSpec((BE8, F), lambda i: (i, 0)),
            pl.BlockSpec((F, 8 * F), lambda i: (0, 0)),
            pl.BlockSpec((1, 8 * F), lambda i: (0, 0)),
        ],
        out_specs=pl.BlockSpec((BE8, 8 * F), lambda i: (i, 0)),
        out_shape=jax.ShapeDtypeStruct((E8, 8 * F), jnp.bfloat16),
    )(edge_attr8, Wbig, bbig)


def _sc_layer(h, e8, src2, dst2):
    mesh = plsc.VectorSubcoreMesh(core_axis_name="c", subcore_axis_name="s")
    EPW8 = EPW // 8

    @functools.partial(
        pl.kernel,
        out_type=jax.ShapeDtypeStruct((NC, N_PAD, F), jnp.float32),
        mesh=mesh,
        scratch_types=[
            pltpu.VMEM((CPG, CH), jnp.int32),
            pltpu.VMEM((CPG, CH), jnp.int32),
            pltpu.VMEM((CH, F), jnp.float32),
            pltpu.VMEM((CH, F), jnp.float32),
            pltpu.VMEM((CH, F), jnp.float32),
            pltpu.VMEM((CH, F // 2), jnp.int32),
            pltpu.VMEM((CH, F // 2), jnp.int32),
            pltpu.VMEM_SHARED((N_PAD, F), jnp.float32),
            pltpu.SemaphoreType.DMA,
            pltpu.SemaphoreType.DMA,
            pltpu.SemaphoreType.DMA,
            pltpu.SemaphoreType.DMA,
            pltpu.SemaphoreType.DMA,
            pltpu.SemaphoreType.DMA,
            pltpu.SemaphoreType.DMA,
            pltpu.SemaphoreType.DMA,
        ],
    )
    def k(h_hbm, e_hbm, src_hbm, dst_hbm, out_hbm,
          src_v, dst_v, rows0, rows1, rows2, e0, e1, agg_sh,
          gsem0, gsem1, gsem2, esem0, esem1, ssem0, ssem1, ssem2):
        cid = lax.axis_index("c")
        sid = lax.axis_index("s")
        wid = cid * NS + sid
        rows = (rows0, rows1, rows2)
        evs = (e0, e1)
        gsems = (gsem0, gsem1, gsem2)
        esems = (esem0, esem1)
        ssems = (ssem0, ssem1, ssem2)

        def zrow(r, carry):
            for c8 in range(F // 16):
                rows0[r, pl.ds(c8 * 16, 16)] = jnp.zeros((16,), jnp.float32)
            return carry
        lax.fori_loop(0, CH, zrow, 0)
        for t in range(RPT // CH):
            pltpu.sync_copy(rows0, agg_sh.at[pl.ds(sid * RPT + t * CH, CH)])
        rem = RPT - (RPT // CH) * CH
        if rem:
            pltpu.sync_copy(rows0.at[pl.ds(0, rem)],
                            agg_sh.at[pl.ds(sid * RPT + (RPT // CH) * CH, rem)])

        plsc.subcore_barrier()

        def start_g(j):
            b = j % GD
            return pltpu.async_copy(h_hbm.at[src_v.at[j]], rows[b], gsems[b])

        def start_e(gg, j):
            b = j % ED
            ebase = jnp.minimum(wid * EPW + (gg * CPG + j) * CH, E - CH)
            return pltpu.async_copy(e_hbm.at[pl.ds(ebase, CH)],
                                    evs[b], esems[b])

        def group(gg, carry):
            base_chunk = wid * CHUNKS + gg * CPG
            pltpu.sync_copy(src_hbm.at[pl.ds(base_chunk, CPG)], src_v)
            pltpu.sync_copy(dst_hbm.at[pl.ds(base_chunk, CPG)], dst_v)

            gpend = [start_g(0), start_g(1)] + [None] * (CPG - 2)
            epend = [start_e(gg, 0)] + [None] * (CPG - 1)
            scat = [None] * GD
            for j in range(CPG):
                if j + 2 < CPG:
                    nb = (j + 2) % GD
                    if scat[nb] is not None:
                        scat[nb].wait()
                        scat[nb] = None
                    gpend[j + 2] = start_g(j + 2)
                if j + 1 < CPG:
                    epend[j + 1] = start_e(gg, j + 1)
                gpend[j].wait()
                epend[j].wait()
                rv, ev = rows[j % GD], evs[j % ED]

                @plsc.parallel_loop(0, CH, unroll=2)
                def crow(r):
                    for c32 in range(F // 32):
                        v = ev[r, pl.ds(c32 * 16, 16)]
                        ea = lax.bitcast_convert_type(v << 16, jnp.float32)
                        eb = lax.bitcast_convert_type(v & jnp.int32(-65536),
                                                      jnp.float32)
                        sa = pl.ds(c32 * 32, 16)
                        sb = pl.ds(c32 * 32 + 16, 16)
                        rv[r, sa] = jnp.maximum(rv[r, sa] + ea, 0.0)
                        rv[r, sb] = jnp.maximum(rv[r, sb] + eb, 0.0)
                scat[j % GD] = pltpu.async_copy(
                    rv, agg_sh.at[dst_v.at[j]], ssems[j % GD], add=True)
            for hnd in scat:
                if hnd is not None:
                    hnd.wait()
            return carry
        lax.fori_loop(0, GROUPS, group, 0)

        plsc.subcore_barrier()
        pltpu.sync_copy(agg_sh.at[pl.ds(sid * RPT, RPT)],
                        out_hbm.at[cid, pl.ds(sid * RPT, RPT)])

    return k(h, e8, src2, dst2)


def _node_mlp_body(h_ref, agg_ref, w1_ref, b1_ref, w2_ref, b2_ref, o_ref):
    z = h_ref[...] + agg_ref[0] + agg_ref[1]
    z1 = jnp.maximum(jnp.dot(z, w1_ref[...], preferred_element_type=jnp.float32)
                     + b1_ref[...], 0.0)
    o_ref[...] = jnp.maximum(
        jnp.dot(z1, w2_ref[...], preferred_element_type=jnp.float32) + b2_ref[...],
        0.0)


def _node_mlp(h, aggs, W1f, b1f, W2, b2):
    return pl.pallas_call(
        _node_mlp_body,
        grid=(NB,),
        in_specs=[
            pl.BlockSpec((BN_, F), lambda i: (i, 0)),
            pl.BlockSpec((NC, BN_, F), lambda i: (0, i, 0)),
            pl.BlockSpec((F, F), lambda i: (0, 0)),
            pl.BlockSpec((1, F), lambda i: (0, 0)),
            pl.BlockSpec((F, F), lambda i: (0, 0)),
            pl.BlockSpec((1, F), lambda i: (0, 0)),
        ],
        out_specs=pl.BlockSpec((BN_, F), lambda i: (i, 0)),
        out_shape=jax.ShapeDtypeStruct((N, F), jnp.float32),
    )(h, aggs, W1f, b1f.reshape(1, F), W2, b2.reshape(1, F))


def _leaky(h):
    return jnp.where(h >= 0, h, 0.01 * h)


def _pool_head_body(h_ref, batch_ref, ge_ref,
                    ow1_ref, ob1_ref, ow2_ref, ob2_ref,
                    gw1_ref, gb1_ref, gw2_ref, gb2_ref,
                    ew1a_ref, ew1b_ref, eb1_ref, ew2_ref, eb2_ref,
                    o_ref, gsum, cnt):
    i = pl.program_id(0)

    @pl.when(i == 0)
    def _():
        gsum[...] = jnp.zeros((G, F), jnp.float32)
        cnt[...] = jnp.zeros((G, 1), jnp.float32)

    b = batch_ref[0, 0, :]
    gids = lax.broadcasted_iota(jnp.int32, (G, BN_), 0)
    oneT = (gids == b[None, :]).astype(jnp.float32)
    gsum[...] += jnp.dot(oneT, h_ref[...], preferred_element_type=jnp.float32)
    cnt[...] += jnp.sum(oneT, axis=1, keepdims=True)

    @pl.when(i == NB - 1)
    def _():
        g = gsum[...] / jnp.maximum(cnt[...], 1.0)
        g = jnp.maximum(jnp.dot(g, ow1_ref[...], preferred_element_type=jnp.float32)
                        + ob1_ref[...], 0.0)
        g = jnp.dot(g, ow2_ref[...], preferred_element_type=jnp.float32) + ob2_ref[...]
        geh = jnp.dot(ge_ref[...], gw1_ref[...], preferred_element_type=jnp.float32) \
            + gb1_ref[...]
        geh = _leaky(geh)
        geh = jnp.dot(geh, gw2_ref[...], preferred_element_type=jnp.float32) \
            + gb2_ref[...]
        z1 = jnp.dot(g, ew1a_ref[...], preferred_element_type=jnp.float32) \
            + jnp.dot(geh, ew1b_ref[...], preferred_element_type=jnp.float32) \
            + eb1_ref[...]
        z1 = _leaky(z1)
        o_ref[...] = jnp.dot(z1, ew2_ref[...], preferred_element_type=jnp.float32) \
            + eb2_ref[...]


def _pool_head(h, batch3, ge, ow1, ob1, ow2, ob2,
               gw1f, gb1f, gw2, gb2, ew1a, ew1b, eb1, ew2, eb2):
    full = lambda *shape: pl.BlockSpec(shape, lambda i: tuple(0 for _ in shape))
    return pl.pallas_call(
        _pool_head_body,
        grid=(NB,),
        in_specs=[
            pl.BlockSpec((BN_, F), lambda i: (i, 0)),
            pl.BlockSpec((1, 1, BN_), lambda i: (i, 0, 0)),
            full(G, 64),
            full(F, F), full(1, F), full(F, F), full(1, F),
            full(64, 64), full(1, 64), full(64, F), full(1, F),
            full(F, 2 * F), full(F, 2 * F), full(1, 2 * F),
            full(2 * F, F), full(1, F),
        ],
        out_specs=pl.BlockSpec((G, F), lambda i: (0, 0)),
        out_shape=jax.ShapeDtypeStruct((G, F), jnp.float32),
        scratch_shapes=[
            pltpu.VMEM((G, F), jnp.float32),
            pltpu.VMEM((G, 1), jnp.float32),
        ],
    )(h, batch3, ge, ow1, ob1.reshape(1, F), ow2, ob2.reshape(1, F),
      gw1f, gb1f.reshape(1, 64), gw2, gb2.reshape(1, F),
      ew1a, ew1b, eb1.reshape(1, 2 * F), ew2, eb2.reshape(1, F))


def kernel(x, edge_index, edge_attr, batch, ge,
           edge_W, edge_b, W1, b1, gamma1, beta1, W2, b2,
           out_W1, out_b1, out_W2, out_b2,
           ge_W1, ge_b1, ge_gamma, ge_beta, ge_W2, ge_b2,
           enc_W1, enc_b1, enc_W2, enc_b2):
    s = 1.0 / jnp.sqrt(1.0 + EPS_BN)
    W1f = W1 * (s * gamma1)[:, None, :]
    b1f = b1 * (s * gamma1) + beta1
    gw1f = ge_W1 * (s * ge_gamma)[None, :]
    gb1f = ge_b1 * (s * ge_gamma) + ge_beta

    pad = E_PAD - E
    src2 = jnp.concatenate([edge_index[0], jnp.zeros((pad,), jnp.int32)]
                           ).reshape(E_PAD // CH, CH)
    dst2 = jnp.concatenate([edge_index[1], jnp.full((pad,), N, jnp.int32)]
                           ).reshape(E_PAD // CH, CH)

    ea8 = edge_attr.reshape(E8, F)
    Wbig = jnp.einsum("ab,lfj->lafbj", jnp.eye(8, dtype=jnp.float32),
                      edge_W[:, :, _PERM]).reshape(L, F, 8 * F)
    bbig = jnp.tile(edge_b[:, _PERM], (1, 8)).reshape(L, 1, 8 * F)
    e8s = [lax.bitcast_convert_type(
        _edge_matmul(ea8, Wbig[li], bbig[li]).reshape(E, F // 2, 2),
        jnp.int32) for li in range(L)]

    h = x
    for li in range(L):
        aggs = _sc_layer(h, e8s[li], src2, dst2)
        h = _node_mlp(h, aggs, W1f[li], b1f[li], W2[li], b2[li])

    batch3 = batch.reshape(NB, 1, BN_)
    return _pool_head(h, batch3, ge,
                      out_W1, out_b1, out_W2, out_b2,
                      gw1f, gb1f, ge_W2, ge_b2,
                      enc_W1[:F], enc_W1[F:], enc_b1, enc_W2, enc_b2)


# speedup vs baseline: 22.7302x; 22.7302x over previous
import functools

import jax
import jax.numpy as jnp
from jax import lax
from jax.experimental import pallas as pl
from jax.experimental.pallas import tpu as pltpu
from jax.experimental.pallas import tpu_sc as plsc

N = 10000
E = 320000
F = 128
G = 128
L = 3
EPS_BN = 1e-5

NC = 2
NS = 16
NW = NC * NS
CH = 64
EPW = 10240
CHUNKS = EPW // CH
CPG = 16
GROUPS = CHUNKS // CPG
E_PAD = NW * EPW
N_PAD = 10112
RPT = N_PAD // NS
GD = 3
ED = 2

BE = 2048
BN_ = 400
NB = N // BN_


E8 = E // 8
BE8 = 200


def _edge_mm_body(a_ref, w_ref, b_ref, o_ref):
    a = a_ref[...]
    o_ref[...] = jnp.dot(a, w_ref[...], preferred_element_type=jnp.float32) \
        + b_ref[...]


def _edge_matmul(edge_attr8, Wbig, bbig):
    return pl.pallas_call(
        _edge_mm_body,
        grid=(E8 // BE8,),
        in_specs=[
            pl.BlockSpec((BE8, F), lambda i: (i, 0)),
            pl.BlockSpec((F, 8 * F), lambda i: (0, 0)),
            pl.BlockSpec((1, 8 * F), lambda i: (0, 0)),
        ],
        out_specs=pl.BlockSpec((BE8, 8 * F), lambda i: (i, 0)),
        out_shape=jax.ShapeDtypeStruct((E8, 8 * F), jnp.float32),
    )(edge_attr8, Wbig, bbig)


def _sc_layer(h, e8, src2, dst2):
    mesh = plsc.VectorSubcoreMesh(core_axis_name="c", subcore_axis_name="s")
    EPW8 = EPW // 8

    @functools.partial(
        pl.kernel,
        out_type=jax.ShapeDtypeStruct((NC, N_PAD, F), jnp.float32),
        mesh=mesh,
        scratch_types=[
            pltpu.VMEM((CPG, CH), jnp.int32),
            pltpu.VMEM((CPG, CH), jnp.int32),
            pltpu.VMEM((CH, F), jnp.float32),
            pltpu.VMEM((CH, F), jnp.float32),
            pltpu.VMEM((CH, F), jnp.float32),
            pltpu.VMEM((CH // 8, 8 * F), jnp.float32),
            pltpu.VMEM((CH // 8, 8 * F), jnp.float32),
            pltpu.VMEM_SHARED((N_PAD, F), jnp.float32),
            pltpu.SemaphoreType.DMA,
            pltpu.SemaphoreType.DMA,
            pltpu.SemaphoreType.DMA,
            pltpu.SemaphoreType.DMA,
            pltpu.SemaphoreType.DMA,
            pltpu.SemaphoreType.DMA,
            pltpu.SemaphoreType.DMA,
            pltpu.SemaphoreType.DMA,
        ],
    )
    def k(h_hbm, e_hbm, src_hbm, dst_hbm, out_hbm,
          src_v, dst_v, rows0, rows1, rows2, e0, e1, agg_sh,
          gsem0, gsem1, gsem2, esem0, esem1, ssem0, ssem1, ssem2):
        cid = lax.axis_index("c")
        sid = lax.axis_index("s")
        wid = cid * NS + sid
        rows = (rows0, rows1, rows2)
        evs = (e0, e1)
        gsems = (gsem0, gsem1, gsem2)
        esems = (esem0, esem1)
        ssems = (ssem0, ssem1, ssem2)

        def zrow(r, carry):
            for c8 in range(F // 16):
                rows0[r, pl.ds(c8 * 16, 16)] = jnp.zeros((16,), jnp.float32)
            return carry
        lax.fori_loop(0, CH, zrow, 0)
        for t in range(RPT // CH):
            pltpu.sync_copy(rows0, agg_sh.at[pl.ds(sid * RPT + t * CH, CH)])
        rem = RPT - (RPT // CH) * CH
        if rem:
            pltpu.sync_copy(rows0.at[pl.ds(0, rem)],
                            agg_sh.at[pl.ds(sid * RPT + (RPT // CH) * CH, rem)])

        plsc.subcore_barrier()

        def start_g(j):
            b = j % GD
            return pltpu.async_copy(h_hbm.at[src_v.at[j]], rows[b], gsems[b])

        def start_e(gg, j):
            b = j % ED
            row8 = jnp.minimum(wid * EPW8 + (gg * CPG + j) * (CH // 8),
                               E8 - CH // 8)
            return pltpu.async_copy(e_hbm.at[pl.ds(row8, CH // 8)],
                                    evs[b], esems[b])

        def group(gg, carry):
            base_chunk = wid * CHUNKS + gg * CPG
            pltpu.sync_copy(src_hbm.at[pl.ds(base_chunk, CPG)], src_v)
            pltpu.sync_copy(dst_hbm.at[pl.ds(base_chunk, CPG)], dst_v)

            gpend = [start_g(0), start_g(1)] + [None] * (CPG - 2)
            epend = [start_e(gg, 0)] + [None] * (CPG - 1)
            scat = [None] * GD
            for j in range(CPG):
                if j + 2 < CPG:
                    nb = (j + 2) % GD
                    if scat[nb] is not None:
                        scat[nb].wait()
                        scat[nb] = None
                    gpend[j + 2] = start_g(j + 2)
                if j + 1 < CPG:
                    epend[j + 1] = start_e(gg, j + 1)
                gpend[j].wait()
                epend[j].wait()
                rv, ev = rows[j % GD], evs[j % ED]

                @plsc.parallel_loop(0, CH, unroll=2)
                def crow(r):
                    rr = r // 8
                    off = (r % 8) * F
                    for c8 in range(F // 16):
                        s = pl.ds(c8 * 16, 16)
                        rv[r, s] = jnp.maximum(
                            rv[r, s] + ev[rr, pl.ds(off + c8 * 16, 16)], 0.0)
                scat[j % GD] = pltpu.async_copy(
                    rv, agg_sh.at[dst_v.at[j]], ssems[j % GD], add=True)
            for hnd in scat:
                if hnd is not None:
                    hnd.wait()
            return carry
        lax.fori_loop(0, GROUPS, group, 0)

        plsc.subcore_barrier()
        pltpu.sync_copy(agg_sh.at[pl.ds(sid * RPT, RPT)],
                        out_hbm.at[cid, pl.ds(sid * RPT, RPT)])

    return k(h, e8, src2, dst2)


def _node_mlp_body(h_ref, agg_ref, w1_ref, b1_ref, w2_ref, b2_ref, o_ref):
    z = h_ref[...] + agg_ref[0] + agg_ref[1]
    z1 = jnp.maximum(jnp.dot(z, w1_ref[...], preferred_element_type=jnp.float32)
                     + b1_ref[...], 0.0)
    o_ref[...] = jnp.maximum(
        jnp.dot(z1, w2_ref[...], preferred_element_type=jnp.float32) + b2_ref[...],
        0.0)


def _node_mlp(h, aggs, W1f, b1f, W2, b2):
    return pl.pallas_call(
        _node_mlp_body,
        grid=(NB,),
        in_specs=[
            pl.BlockSpec((BN_, F), lambda i: (i, 0)),
            pl.BlockSpec((NC, BN_, F), lambda i: (0, i, 0)),
            pl.BlockSpec((F, F), lambda i: (0, 0)),
            pl.BlockSpec((1, F), lambda i: (0, 0)),
            pl.BlockSpec((F, F), lambda i: (0, 0)),
            pl.BlockSpec((1, F), lambda i: (0, 0)),
        ],
        out_specs=pl.BlockSpec((BN_, F), lambda i: (i, 0)),
        out_shape=jax.ShapeDtypeStruct((N, F), jnp.float32),
    )(h, aggs, W1f, b1f.reshape(1, F), W2, b2.reshape(1, F))


def _leaky(h):
    return jnp.where(h >= 0, h, 0.01 * h)


def _pool_head_body(h_ref, batch_ref, ge_ref,
                    ow1_ref, ob1_ref, ow2_ref, ob2_ref,
                    gw1_ref, gb1_ref, gw2_ref, gb2_ref,
                    ew1a_ref, ew1b_ref, eb1_ref, ew2_ref, eb2_ref,
                    o_ref, gsum, cnt):
    i = pl.program_id(0)

    @pl.when(i == 0)
    def _():
        gsum[...] = jnp.zeros((G, F), jnp.float32)
        cnt[...] = jnp.zeros((G, 1), jnp.float32)

    b = batch_ref[0, 0, :]
    gids = lax.broadcasted_iota(jnp.int32, (G, BN_), 0)
    oneT = (gids == b[None, :]).astype(jnp.float32)
    gsum[...] += jnp.dot(oneT, h_ref[...], preferred_element_type=jnp.float32)
    cnt[...] += jnp.sum(oneT, axis=1, keepdims=True)

    @pl.when(i == NB - 1)
    def _():
        g = gsum[...] / jnp.maximum(cnt[...], 1.0)
        g = jnp.maximum(jnp.dot(g, ow1_ref[...], preferred_element_type=jnp.float32)
                        + ob1_ref[...], 0.0)
        g = jnp.dot(g, ow2_ref[...], preferred_element_type=jnp.float32) + ob2_ref[...]
        geh = jnp.dot(ge_ref[...], gw1_ref[...], preferred_element_type=jnp.float32) \
            + gb1_ref[...]
        geh = _leaky(geh)
        geh = jnp.dot(geh, gw2_ref[...], preferred_element_type=jnp.float32) \
            + gb2_ref[...]
        z1 = jnp.dot(g, ew1a_ref[...], preferred_element_type=jnp.float32) \
            + jnp.dot(geh, ew1b_ref[...], preferred_element_type=jnp.float32) \
            + eb1_ref[...]
        z1 = _leaky(z1)
        o_ref[...] = jnp.dot(z1, ew2_ref[...], preferred_element_type=jnp.float32) \
            + eb2_ref[...]


def _pool_head(h, batch3, ge, ow1, ob1, ow2, ob2,
               gw1f, gb1f, gw2, gb2, ew1a, ew1b, eb1, ew2, eb2):
    full = lambda *shape: pl.BlockSpec(shape, lambda i: tuple(0 for _ in shape))
    return pl.pallas_call(
        _pool_head_body,
        grid=(NB,),
        in_specs=[
            pl.BlockSpec((BN_, F), lambda i: (i, 0)),
            pl.BlockSpec((1, 1, BN_), lambda i: (i, 0, 0)),
            full(G, 64),
            full(F, F), full(1, F), full(F, F), full(1, F),
            full(64, 64), full(1, 64), full(64, F), full(1, F),
            full(F, 2 * F), full(F, 2 * F), full(1, 2 * F),
            full(2 * F, F), full(1, F),
        ],
        out_specs=pl.BlockSpec((G, F), lambda i: (0, 0)),
        out_shape=jax.ShapeDtypeStruct((G, F), jnp.float32),
        scratch_shapes=[
            pltpu.VMEM((G, F), jnp.float32),
            pltpu.VMEM((G, 1), jnp.float32),
        ],
    )(h, batch3, ge, ow1, ob1.reshape(1, F), ow2, ob2.reshape(1, F),
      gw1f, gb1f.reshape(1, 64), gw2, gb2.reshape(1, F),
      ew1a, ew1b, eb1.reshape(1, 2 * F), ew2, eb2.reshape(1, F))


def kernel(x, edge_index, edge_attr, batch, ge,
           edge_W, edge_b, W1, b1, gamma1, beta1, W2, b2,
           out_W1, out_b1, out_W2, out_b2,
           ge_W1, ge_b1, ge_gamma, ge_beta, ge_W2, ge_b2,
           enc_W1, enc_b1, enc_W2, enc_b2):
    s = 1.0 / jnp.sqrt(1.0 + EPS_BN)
    W1f = W1 * (s * gamma1)[:, None, :]
    b1f = b1 * (s * gamma1) + beta1
    gw1f = ge_W1 * (s * ge_gamma)[None, :]
    gb1f = ge_b1 * (s * ge_gamma) + ge_beta

    pad = E_PAD - E
    src2 = jnp.concatenate([edge_index[0], jnp.zeros((pad,), jnp.int32)]
                           ).reshape(E_PAD // CH, CH)
    dst2 = jnp.concatenate([edge_index[1], jnp.full((pad,), N, jnp.int32)]
                           ).reshape(E_PAD // CH, CH)

    ea8 = edge_attr.reshape(E8, F)
    Wbig = jnp.einsum("ab,lfj->lafbj", jnp.eye(8, dtype=jnp.float32),
                      edge_W).reshape(L, F, 8 * F)
    bbig = jnp.tile(edge_b, (1, 8)).reshape(L, 1, 8 * F)
    e8s = [_edge_matmul(ea8, Wbig[li], bbig[li]) for li in range(L)]

    h = x
    for li in range(L):
        aggs = _sc_layer(h, e8s[li], src2, dst2)
        h = _node_mlp(h, aggs, W1f[li], b1f[li], W2[li], b2[li])

    batch3 = batch.reshape(NB, 1, BN_)
    return _pool_head(h, batch3, ge,
                      out_W1, out_b1, out_W2, out_b2,
                      gw1f, gb1f, ge_W2, ge_b2,
                      enc_W1[:F], enc_W1[F:], enc_b1, enc_W2, enc_b2)
